# baseline (speedup 1.0000x reference)
import functools

import jax
import jax.numpy as jnp
from jax import lax
from jax.experimental import pallas as pl
from jax.experimental.pallas import tpu as pltpu
from jax.experimental.pallas import tpu_sc as plsc

NUM_POSITIONS = 8192
EMBEDDING_DIM = 2048
TOTAL = 4 * 8192

NUM_WORKERS = 32
R = 16
NBLK = NUM_POSITIONS // R
NB = NBLK // NUM_WORKERS
W = 256
PADDED = TOTAL + W
SW = 32
ROW_BYTES = EMBEDDING_DIM * 4


def _extract(vec_ref, j):
    base = pl.multiple_of((j // 16) * 16, 8)
    v = vec_ref[pl.ds(base, 16)]
    lane = j - base
    return jnp.sum(jnp.where(lax.iota(jnp.int32, 16) == lane, v, 0))


def _emb_body(sidx_hbm, order_hbm, starts_hbm, table_hbm, out_hbm,
              bufs, sidxw, orderw, startsw, rsems, wsem):
    nc = plsc.get_sparse_core_info().num_cores
    wid = lax.axis_index("s") * nc + lax.axis_index("c")
    blk0 = wid * NB

    pltpu.sync_copy(starts_hbm.at[pl.ds(blk0, SW)], startsw)

    def read(b_loc, buf):
        return pltpu.make_async_copy(
            table_hbm.at[pl.ds((blk0 + b_loc) * R, R)], bufs.at[buf],
            rsems.at[buf]
        )

    def wdrain():
        pltpu.make_async_copy(
            table_hbm.at[pl.ds(0, 1)], bufs.at[0, pl.ds(0, 1)], wsem
        ).wait()

    def process_block(b_loc, buf, s_lo, s_hi):
        row_base = (blk0 + b_loc) * R

        def window(j0):
            wb = pl.multiple_of((j0 // 8) * 8, 8)
            pltpu.sync_copy(sidx_hbm.at[pl.ds(wb, W)], sidxw)
            pltpu.sync_copy(order_hbm.at[pl.ds(wb, W)], orderw)
            jend = jnp.minimum(s_hi, wb + W - 16)

            def entry(j, _):
                src = _extract(sidxw, j - wb) - row_base
                dst = _extract(orderw, j - wb)
                pltpu.make_async_copy(
                    bufs.at[buf, pl.ds(src, 1)], out_hbm.at[pl.ds(dst, 1)],
                    wsem
                ).start()
                return 0

            lax.fori_loop(j0, jend, entry, 0)
            return jend

        lax.while_loop(lambda j: j < s_hi, window, s_lo)

    read(0, 0).start()

    def body(b_loc, prev_n):
        buf = lax.rem(b_loc, 2)
        s_lo = _extract(startsw, b_loc)
        s_hi = _extract(startsw, b_loc + 1)

        lax.fori_loop(0, prev_n, lambda _, c: (wdrain(), c)[1], 0)

        @pl.when(b_loc + 1 < NB)
        def _():
            read(b_loc + 1, 1 - buf).start()

        read(b_loc, buf).wait()
        process_block(b_loc, buf, s_lo, s_hi)
        return s_hi - s_lo

    last_n = lax.fori_loop(0, NB, body, 0)
    lax.fori_loop(0, last_n, lambda _, c: (wdrain(), c)[1], 0)


@functools.partial(
    pl.kernel,
    out_type=jax.ShapeDtypeStruct((TOTAL, EMBEDDING_DIM), jnp.float32),
    mesh=plsc.VectorSubcoreMesh(core_axis_name="c", subcore_axis_name="s"),
    compiler_params=pltpu.CompilerParams(needs_layout_passes=False),
    scratch_types=[
        pltpu.VMEM((2, R, EMBEDDING_DIM), jnp.float32),
        pltpu.VMEM((W,), jnp.int32),
        pltpu.VMEM((W,), jnp.int32),
        pltpu.VMEM((SW,), jnp.int32),
        pltpu.SemaphoreType.DMA((2,)),
        pltpu.SemaphoreType.DMA,
    ],
)
def _emb(sidx_hbm, order_hbm, starts_hbm, table_hbm, out_hbm,
         bufs, sidxw, orderw, startsw, rsems, wsem):
    _emb_body(sidx_hbm, order_hbm, starts_hbm, table_hbm, out_hbm,
              bufs, sidxw, orderw, startsw, rsems, wsem)


def kernel(positions, weight):
    flat = positions.reshape(-1)
    order = jnp.argsort(flat).astype(jnp.int32)
    sidx = jnp.take(flat, order).astype(jnp.int32)
    starts = jnp.searchsorted(
        sidx, jnp.arange(0, NUM_POSITIONS + 1, R, dtype=jnp.int32)
    ).astype(jnp.int32)
    sidx_p = jnp.pad(sidx, (0, PADDED - TOTAL))
    order_p = jnp.pad(order, (0, PADDED - TOTAL))
    starts_p = jnp.pad(starts, (0, NBLK + SW - starts.shape[0]),
                       constant_values=TOTAL)
    out = _emb(sidx_p, order_p, starts_p, weight)
    return out.reshape(positions.shape + (weight.shape[1],))


# speedup vs baseline: 1.0961x; 1.0961x over previous
import functools

import jax
import jax.numpy as jnp
from jax import lax
from jax.experimental import pallas as pl
from jax.experimental.pallas import tpu as pltpu
from jax.experimental.pallas import tpu_sc as plsc

NUM_POSITIONS = 8192
EMBEDDING_DIM = 2048
TOTAL = 4 * 8192

NUM_WORKERS = 32
B_PER_W = TOTAL // NUM_WORKERS
K = 8
NBUF = 5
G = NBUF - 1
STEPS = B_PER_W // K


def _emb_body(idx_hbm, table_hbm, out_hbm, idx_v, rows_v, gsems, osems):
    nc = plsc.get_sparse_core_info().num_cores
    wid = lax.axis_index("s") * nc + lax.axis_index("c")
    base = wid * B_PER_W

    pltpu.sync_copy(idx_hbm.at[pl.ds(base, B_PER_W)], idx_v)

    def gather(step, buf):
        off = pl.multiple_of(step * K, 8)
        return pltpu.make_async_copy(
            table_hbm.at[idx_v.at[pl.ds(off, K)]], rows_v.at[buf], gsems.at[buf]
        )

    def write(step, buf):
        off = pl.multiple_of(base + step * K, 8)
        return pltpu.make_async_copy(
            rows_v.at[buf], out_hbm.at[pl.ds(off, K)], osems.at[buf]
        )

    for b in range(G):
        gather(b, b).start()

    def body(i, _):
        buf = lax.rem(i, NBUF)

        @pl.when(i + G < STEPS)
        def _():
            nbuf = lax.rem(i + G, NBUF)

            @pl.when(i >= 1)
            def _():
                write(i - 1, nbuf).wait()

            gather(i + G, nbuf).start()

        gather(i, buf).wait()
        write(i, buf).start()
        return 0

    lax.fori_loop(0, STEPS, body, 0)

    for j in range(STEPS - G - 1, STEPS):
        write(j, j % NBUF).wait()


@functools.partial(
    pl.kernel,
    out_type=jax.ShapeDtypeStruct((TOTAL, EMBEDDING_DIM), jnp.float32),
    mesh=plsc.VectorSubcoreMesh(core_axis_name="c", subcore_axis_name="s"),
    scratch_types=[
        pltpu.VMEM((B_PER_W,), jnp.int32),
        pltpu.VMEM((NBUF, K, EMBEDDING_DIM), jnp.float32),
        pltpu.SemaphoreType.DMA((NBUF,)),
        pltpu.SemaphoreType.DMA((NBUF,)),
    ],
)
def _emb(idx_hbm, table_hbm, out_hbm, idx_v, rows_v, gsems, osems):
    _emb_body(idx_hbm, table_hbm, out_hbm, idx_v, rows_v, gsems, osems)


def kernel(positions, weight):
    flat = positions.reshape(-1)
    out = _emb(flat, weight)
    return out.reshape(positions.shape + (weight.shape[1],))
